# Initial kernel scaffold; baseline (speedup 1.0000x reference)
import jax
import jax.numpy as jnp
from jax.experimental import pallas as pl


def kernel(logits, sim_matrix, targets):
    raise NotImplementedError("write your pallas kernel here")


# speedup vs baseline: 1.7056x; 1.7056x over previous
import functools

import jax
import jax.numpy as jnp
from jax import lax
from jax.experimental import pallas as pl
from jax.experimental.pallas import tpu as pltpu
from jax.experimental.pallas import tpu_sc as plsc

V = 8192
D = 8192
R = 2048
NC = 2
NS = 16
NW = NC * NS
RPW = R // NW
CH = 2
NCH = RPW // CH
L = 16
NBUF = 2


def _sc_body(x_hbm, sim_hbm, idx_hbm, out_hbm,
             idx_v, simbuf, xbuf, res,
             sem_s0, sem_s1, sem_x0, sem_x1):
    sem_s = (sem_s0, sem_s1)
    sem_x = (sem_x0, sem_x1)
    wid = lax.axis_index("s") * NC + lax.axis_index("c")
    base = wid * RPW

    pltpu.sync_copy(idx_hbm.at[wid], idx_v)

    def issue(c, buf):
        pltpu.async_copy(sim_hbm.at[idx_v.at[c]], simbuf.at[buf], sem_s[buf])
        pltpu.async_copy(x_hbm.at[pl.ds(base + c * CH, CH)], xbuf.at[buf],
                         sem_x[buf])

    def wait_chunk(c, buf):
        pltpu.make_async_copy(sim_hbm.at[idx_v.at[c]], simbuf.at[buf],
                              sem_s[buf]).wait()
        pltpu.make_async_copy(x_hbm.at[pl.ds(base + c * CH, CH)],
                              xbuf.at[buf], sem_x[buf]).wait()

    def compute(c, buf):
        for r in range(CH):
            row = c * CH + r

            def body(i, carry):
                sxr, sxx, srr = carry
                off = i * L
                xv = xbuf[buf, r, pl.ds(off, L)]
                rv = simbuf[buf, r, pl.ds(off, L)]
                return (sxr + xv * rv, sxx + xv * xv, srr + rv * rv)

            z = jnp.zeros((L,), jnp.float32)
            sxr, sxx, srr = lax.fori_loop(0, D // L, body, (z, z, z),
                                          unroll=8)
            res[0, row] = sxr
            res[1, row] = sxx
            res[2, row] = srr

    issue(0, 0)
    issue(1, 1)
    for c in range(NCH):
        buf = c % NBUF
        wait_chunk(c, buf)
        compute(c, buf)
        if c + NBUF < NCH:
            issue(c + NBUF, buf)

    pltpu.sync_copy(res, out_hbm.at[wid])


@functools.partial(
    pl.kernel,
    out_type=jax.ShapeDtypeStruct((NW, 3, RPW, L), jnp.float32),
    mesh=plsc.VectorSubcoreMesh(core_axis_name="c", subcore_axis_name="s"),
    scratch_types=[
        pltpu.VMEM((NCH, CH), jnp.int32),
        pltpu.VMEM((NBUF, CH, D), jnp.float32),
        pltpu.VMEM((NBUF, CH, D), jnp.float32),
        pltpu.VMEM((3, RPW, L), jnp.float32),
        pltpu.SemaphoreType.DMA,
        pltpu.SemaphoreType.DMA,
        pltpu.SemaphoreType.DMA,
        pltpu.SemaphoreType.DMA,
    ],
    name="similarity_gather_dot_sc",
)
def _sc_gather_dot(x_hbm, sim_hbm, idx_hbm, out_hbm, *scratch):
    _sc_body(x_hbm, sim_hbm, idx_hbm, out_hbm, *scratch)


def _epilogue_body(p_ref, o_ref):
    p = p_ref[...]
    s = jnp.sum(p, axis=-1)
    dot = s[:, 0, :]
    sxx = s[:, 1, :]
    srr = s[:, 2, :]
    x_norm = jnp.sqrt(sxx)
    nx = jnp.maximum(x_norm, 1e-12)
    num = dot / nx
    xn_norm = x_norm / nx
    den = jnp.maximum(xn_norm * jnp.sqrt(srr), 1e-8)
    cos = num / den
    o_ref[0, 0] = jnp.mean(1.0 - cos)


def kernel(logits, sim_matrix, targets):
    x = logits.reshape(R, D)
    t = targets.reshape(-1).astype(jnp.int32).reshape(NW, NCH, CH)
    part = _sc_gather_dot(x, sim_matrix, t)
    loss = pl.pallas_call(
        _epilogue_body,
        out_shape=jax.ShapeDtypeStruct((1, 1), jnp.float32),
        out_specs=pl.BlockSpec(memory_space=pltpu.SMEM),
        name="similarity_loss_epilogue_tc",
    )(part)
    return loss[0, 0]
